# Initial kernel scaffold; baseline (speedup 1.0000x reference)
import jax
import jax.numpy as jnp
from jax.experimental import pallas as pl


def kernel(atomic_basis, c_tilde_indices, c_tilde_values, a_update0_indices, a_update0_values, a_update1_indices, a_update1_values):
    raise NotImplementedError("write your pallas kernel here")


# speedup vs baseline: 10.3554x; 10.3554x over previous
import functools

import jax
import jax.numpy as jnp
from jax import lax
from jax.experimental import pallas as pl
from jax.experimental.pallas import tpu as pltpu
from jax.experimental.pallas import tpu_sc as plsc

N = 16384
D = 64
_NNZ_C = 268435
_NNZ_U = 65536

_NUM_SUBCORES = 16
_LANES = 16
_DH = D // 2

_WIN = 128
_WPG = 8
_GROUP = _WIN * _WPG
_ROWS_PER_TILE = N // _NUM_SUBCORES

_R_RAW = _NNZ_C + _NNZ_U
_R_GROUPS = -(-_R_RAW // (_NUM_SUBCORES * _GROUP))
_R_PAD = _NUM_SUBCORES * _R_GROUPS * _GROUP
_I_GROUPS = _NNZ_U // (_NUM_SUBCORES * _GROUP)

_mesh = plsc.VectorSubcoreMesh(core_axis_name="c", subcore_axis_name="s")


@functools.partial(
    pl.kernel,
    out_type=(
        jax.ShapeDtypeStruct((N, D), jnp.float32),
        jax.ShapeDtypeStruct((N, D), jnp.float32),
    ),
    mesh=_mesh,
    scratch_types=(
        pltpu.VMEM((_WPG, _WIN), jnp.int32),
        pltpu.VMEM((_WPG, _WIN), jnp.int32),
        pltpu.VMEM((_GROUP,), jnp.float32),
        pltpu.VMEM((_GROUP, _DH), jnp.float32),
        pltpu.VMEM_SHARED((N, _DH), jnp.float32),
        pltpu.SemaphoreType.DMA,
        pltpu.SemaphoreType.DMA,
    ),
    compiler_params=pltpu.CompilerParams(use_tc_tiling_on_sc=False),
)
def _sc_spmm(basis_lo, basis_hi, rr, rc, rv, ir, ic, iv, out_r, out_i,
             rows_b, cols_b, vals_b, gath, acc, gsem, ssem):
    cid = lax.axis_index("c")
    sid = lax.axis_index("s")
    rsl = pl.ds(sid * _ROWS_PER_TILE, _ROWS_PER_TILE)
    zero16 = jnp.zeros((_LANES,), jnp.float32)

    def zero_acc_slice():
        def zrow(r, carry):
            for c in range(_DH // _LANES):
                gath[r, pl.ds(c * _LANES, _LANES)] = zero16
            return carry

        lax.fori_loop(0, _GROUP, zrow, 0)
        pltpu.sync_copy(gath, acc.at[rsl, :])

    def process(basis, rows2, cols2, vflat, ngroups):
        def grp(g, carry):
            gbase = sid * ngroups + g
            base = gbase * _GROUP
            wb = gbase * _WPG
            pltpu.sync_copy(rows2.at[pl.ds(wb, _WPG), :], rows_b)
            pltpu.sync_copy(cols2.at[pl.ds(wb, _WPG), :], cols_b)
            pltpu.sync_copy(vflat.at[pl.ds(base, _GROUP)], vals_b)
            cps = [
                pltpu.async_copy(
                    basis.at[cols_b.at[j]],
                    gath.at[pl.ds(j * _WIN, _WIN), :], gsem)
                for j in range(_WPG)
            ]
            for cp in cps:
                cp.wait()

            def srow16(r0, inner):
                v16 = vals_b[pl.ds(r0 * _LANES, _LANES)]
                for l in range(_LANES):
                    r = r0 * _LANES + l
                    vv = jnp.broadcast_to(v16[l], (_LANES,))
                    for c in range(_DH // _LANES):
                        sl = pl.ds(c * _LANES, _LANES)
                        gath[r, sl] = gath[r, sl] * vv
                return inner

            lax.fori_loop(0, _GROUP // _LANES, srow16, 0)
            scps = [
                pltpu.async_copy(
                    gath.at[pl.ds(j * _WIN, _WIN), :],
                    acc.at[rows_b.at[j]], ssem, add=True)
                for j in range(_WPG)
            ]
            for cp in scps:
                cp.wait()
            return carry

        lax.fori_loop(0, ngroups, grp, 0)

    def full_flow(basis, colofs):
        csl = pl.ds(colofs, _DH)
        zero_acc_slice()
        plsc.subcore_barrier()
        process(basis, rr, rc, rv, _R_GROUPS)
        plsc.subcore_barrier()
        pltpu.sync_copy(acc.at[rsl, :], gath)
        pltpu.sync_copy(gath, out_r.at[rsl, csl])
        zero_acc_slice()
        plsc.subcore_barrier()
        process(basis, ir, ic, iv, _I_GROUPS)
        plsc.subcore_barrier()
        pltpu.sync_copy(acc.at[rsl, :], gath)
        pltpu.sync_copy(gath, out_i.at[rsl, csl])

    @pl.when(cid == 0)
    def _():
        full_flow(basis_lo, 0)

    @pl.when(cid == 1)
    def _():
        full_flow(basis_hi, _DH)


def kernel(atomic_basis, c_tilde_indices, c_tilde_values,
           a_update0_indices, a_update0_values,
           a_update1_indices, a_update1_values):
    rows_r = jnp.concatenate([c_tilde_indices[1], a_update0_indices[0]])
    cols_r = jnp.concatenate([c_tilde_indices[0], a_update0_indices[1]])
    vals_r = jnp.concatenate([c_tilde_values, a_update0_values])
    npad = _R_PAD - _R_RAW
    pad_idx = (jnp.arange(npad, dtype=jnp.int32) % N).astype(jnp.int32)
    rows_r = jnp.concatenate([rows_r.astype(jnp.int32), pad_idx])
    cols_r = jnp.concatenate([cols_r.astype(jnp.int32), pad_idx])
    vals_r = jnp.concatenate([vals_r, jnp.zeros((npad,), jnp.float32)])

    rr = rows_r.reshape(_R_PAD // _WIN, _WIN)
    rc = cols_r.reshape(_R_PAD // _WIN, _WIN)
    ir = a_update1_indices[0].astype(jnp.int32).reshape(_NNZ_U // _WIN, _WIN)
    ic = a_update1_indices[1].astype(jnp.int32).reshape(_NNZ_U // _WIN, _WIN)

    basis_lo = atomic_basis[:, :_DH]
    basis_hi = atomic_basis[:, _DH:]

    real_out, imag_out = _sc_spmm(
        basis_lo, basis_hi, rr, rc, vals_r, ir, ic, a_update1_values)
    return (real_out, imag_out)


# speedup vs baseline: 10.9732x; 1.0597x over previous
import functools

import jax
import jax.numpy as jnp
from jax import lax
from jax.experimental import pallas as pl
from jax.experimental.pallas import tpu as pltpu
from jax.experimental.pallas import tpu_sc as plsc

N = 16384
D = 64
_NNZ_C = 268435
_NNZ_U = 65536

_NUM_SUBCORES = 16
_LANES = 16
_DH = D // 2

_WIN = 128
_WPG = 8
_GROUP = _WIN * _WPG
_ROWS_PER_TILE = N // _NUM_SUBCORES

_R_RAW = _NNZ_C + _NNZ_U
_R_GROUPS = -(-_R_RAW // (_NUM_SUBCORES * _GROUP))
_R_PAD = _NUM_SUBCORES * _R_GROUPS * _GROUP
_I_GROUPS = _NNZ_U // (_NUM_SUBCORES * _GROUP)

_mesh = plsc.VectorSubcoreMesh(core_axis_name="c", subcore_axis_name="s")


@functools.partial(
    pl.kernel,
    out_type=(
        jax.ShapeDtypeStruct((N, _DH), jnp.float32),
        jax.ShapeDtypeStruct((N, _DH), jnp.float32),
        jax.ShapeDtypeStruct((N, _DH), jnp.float32),
        jax.ShapeDtypeStruct((N, _DH), jnp.float32),
    ),
    mesh=_mesh,
    scratch_types=(
        pltpu.VMEM((2, _WPG, _WIN), jnp.int32),
        pltpu.VMEM((2, _WPG, _WIN), jnp.int32),
        pltpu.VMEM((2, _GROUP), jnp.float32),
        pltpu.VMEM((_GROUP, _DH), jnp.float32),
        pltpu.VMEM((_GROUP, _DH), jnp.float32),
        pltpu.VMEM_SHARED((N, _DH), jnp.float32),
        pltpu.SemaphoreType.DMA,
        pltpu.SemaphoreType.DMA,
        pltpu.SemaphoreType.DMA,
        pltpu.SemaphoreType.DMA,
    ),
    compiler_params=pltpu.CompilerParams(use_tc_tiling_on_sc=False),
)
def _sc_spmm(basis_lo, basis_hi, rr, rc, rv, ir, ic, iv,
             out_rl, out_rh, out_il, out_ih,
             rows_b, cols_b, vals_b, gath0, gath1, acc,
             gsem0, gsem1, ssem0, ssem1):
    cid = lax.axis_index("c")
    sid = lax.axis_index("s")
    rsl = pl.ds(sid * _ROWS_PER_TILE, _ROWS_PER_TILE)
    zero16 = jnp.zeros((_LANES,), jnp.float32)
    gaths = (gath0, gath1)
    gsems = (gsem0, gsem1)
    ssems = (ssem0, ssem1)

    def zero_acc_slice():
        def zrow(r, carry):
            for c in range(_DH // _LANES):
                gath0[r, pl.ds(c * _LANES, _LANES)] = zero16
            return carry

        lax.fori_loop(0, _GROUP, zrow, 0)
        pltpu.sync_copy(gath0, acc.at[rsl, :])

    def process(basis, rows2, cols2, vflat, ngroups):
        def drain(sem):
            pltpu.make_async_copy(
                basis.at[pl.ds(0, _GROUP), :], gaths[0], sem).wait()

        def fetch(b, g, first):
            @pl.when(g < ngroups)
            def _():
                if not first:
                    drain(ssems[b])
                gbase = sid * ngroups + g
                pltpu.sync_copy(rows2.at[pl.ds(gbase * _WPG, _WPG), :],
                                rows_b.at[b])
                pltpu.sync_copy(cols2.at[pl.ds(gbase * _WPG, _WPG), :],
                                cols_b.at[b])
                pltpu.sync_copy(vflat.at[pl.ds(gbase * _GROUP, _GROUP)],
                                vals_b.at[b])
                for j in range(_WPG):
                    pltpu.async_copy(
                        basis.at[cols_b.at[b, j]],
                        gaths[b].at[pl.ds(j * _WIN, _WIN), :], gsems[b])

        def compute(b, g):
            @pl.when(g < ngroups)
            def _():
                drain(gsems[b])
                gath = gaths[b]

                def srow16(r0, inner):
                    v16 = vals_b[b, pl.ds(r0 * _LANES, _LANES)]
                    for l in range(_LANES):
                        r = r0 * _LANES + l
                        vv = jnp.broadcast_to(v16[l], (_LANES,))
                        for c in range(_DH // _LANES):
                            sl = pl.ds(c * _LANES, _LANES)
                            gath[r, sl] = gath[r, sl] * vv
                    return inner

                lax.fori_loop(0, _GROUP // _LANES, srow16, 0)
                for j in range(_WPG):
                    pltpu.async_copy(
                        gath.at[pl.ds(j * _WIN, _WIN), :],
                        acc.at[rows_b.at[b, j]], ssems[b], add=True)

        fetch(0, 0, first=True)
        fetch(1, 1, first=True)

        def pair(p, carry):
            for b in (0, 1):
                g = 2 * p + b
                compute(b, g)
                fetch(b, g + 2, first=False)
            return carry

        lax.fori_loop(0, (ngroups + 1) // 2, pair, 0)
        drain(ssems[0])
        drain(ssems[1])

    def full_flow(basis, out_r, out_i):
        zero_acc_slice()
        plsc.subcore_barrier()
        process(basis, rr, rc, rv, _R_GROUPS)
        plsc.subcore_barrier()
        pltpu.sync_copy(acc.at[rsl, :], gath0)
        pltpu.sync_copy(gath0, out_r.at[rsl, :])
        zero_acc_slice()
        plsc.subcore_barrier()
        process(basis, ir, ic, iv, _I_GROUPS)
        plsc.subcore_barrier()
        pltpu.sync_copy(acc.at[rsl, :], gath0)
        pltpu.sync_copy(gath0, out_i.at[rsl, :])

    @pl.when(cid == 0)
    def _():
        full_flow(basis_lo, out_rl, out_il)

    @pl.when(cid == 1)
    def _():
        full_flow(basis_hi, out_rh, out_ih)


def kernel(atomic_basis, c_tilde_indices, c_tilde_values,
           a_update0_indices, a_update0_values,
           a_update1_indices, a_update1_values):
    rows_r = jnp.concatenate([c_tilde_indices[1], a_update0_indices[0]])
    cols_r = jnp.concatenate([c_tilde_indices[0], a_update0_indices[1]])
    vals_r = jnp.concatenate([c_tilde_values, a_update0_values])
    npad = _R_PAD - _R_RAW
    pad_idx = (jnp.arange(npad, dtype=jnp.int32) % N).astype(jnp.int32)
    rows_r = jnp.concatenate([rows_r.astype(jnp.int32), pad_idx])
    cols_r = jnp.concatenate([cols_r.astype(jnp.int32), pad_idx])
    vals_r = jnp.concatenate([vals_r, jnp.zeros((npad,), jnp.float32)])

    rr = rows_r.reshape(_R_PAD // _WIN, _WIN)
    rc = cols_r.reshape(_R_PAD // _WIN, _WIN)
    ir = a_update1_indices[0].astype(jnp.int32).reshape(_NNZ_U // _WIN, _WIN)
    ic = a_update1_indices[1].astype(jnp.int32).reshape(_NNZ_U // _WIN, _WIN)

    basis_lo = atomic_basis[:, :_DH]
    basis_hi = atomic_basis[:, _DH:]

    out_rl, out_rh, out_il, out_ih = _sc_spmm(
        basis_lo, basis_hi, rr, rc, vals_r, ir, ic, a_update1_values)
    real_out = jnp.concatenate([out_rl, out_rh], axis=1)
    imag_out = jnp.concatenate([out_il, out_ih], axis=1)
    return (real_out, imag_out)


# speedup vs baseline: 11.1543x; 1.0165x over previous
import functools

import jax
import jax.numpy as jnp
from jax import lax
from jax.experimental import pallas as pl
from jax.experimental.pallas import tpu as pltpu
from jax.experimental.pallas import tpu_sc as plsc

N = 16384
D = 64
_NNZ_C = 268435
_NNZ_U = 65536

_NUM_SUBCORES = 16
_LANES = 16
_DH = D // 2

_WIN = 128
_WPG = 8
_GROUP = _WIN * _WPG
_ROWS_PER_TILE = N // _NUM_SUBCORES

_CT_WINS = _NNZ_C // _WIN
_CT_FULL = _CT_WINS * _WIN
_CT_TAIL = _NNZ_C - _CT_FULL
_A0X_LEN = _NNZ_U + _WIN
_A0X_WINS = _A0X_LEN // _WIN
_A1_WINS = _NNZ_U // _WIN

_mesh = plsc.VectorSubcoreMesh(core_axis_name="c", subcore_axis_name="s")


@functools.partial(
    pl.kernel,
    out_type=(
        jax.ShapeDtypeStruct((N, _DH), jnp.float32),
        jax.ShapeDtypeStruct((N, _DH), jnp.float32),
        jax.ShapeDtypeStruct((N, _DH), jnp.float32),
        jax.ShapeDtypeStruct((N, _DH), jnp.float32),
    ),
    mesh=_mesh,
    scratch_types=(
        pltpu.VMEM((2, _WPG, _WIN), jnp.int32),
        pltpu.VMEM((2, _WPG, _WIN), jnp.int32),
        pltpu.VMEM((2, _GROUP), jnp.float32),
        pltpu.VMEM((_GROUP, _DH), jnp.float32),
        pltpu.VMEM((_GROUP, _DH), jnp.float32),
        pltpu.VMEM_SHARED((N, _DH), jnp.float32),
        pltpu.SemaphoreType.DMA,
        pltpu.SemaphoreType.DMA,
        pltpu.SemaphoreType.DMA,
        pltpu.SemaphoreType.DMA,
        pltpu.SemaphoreType.DMA,
    ),
    compiler_params=pltpu.CompilerParams(use_tc_tiling_on_sc=False),
)
def _sc_spmm(basis_lo, basis_hi, ct_rows, ct_cols, ct_vals,
             a0x_rows, a0x_cols, a0x_vals, a1_rows, a1_cols, a1_vals,
             out_rl, out_rh, out_il, out_ih,
             rows_b, cols_b, vals_b, gath0, gath1, acc,
             gsem0, gsem1, ssem0, ssem1, isem):
    cid = lax.axis_index("c")
    sid = lax.axis_index("s")
    rsl = pl.ds(sid * _ROWS_PER_TILE, _ROWS_PER_TILE)
    zero16f = jnp.zeros((_LANES,), jnp.float32)
    gaths = (gath0, gath1)
    gsems = (gsem0, gsem1)
    ssems = (ssem0, ssem1)

    def zero_acc_slice():
        def zrow(r, carry):
            for c in range(_DH // _LANES):
                gath0[r, pl.ds(c * _LANES, _LANES)] = zero16f
            return carry

        lax.fori_loop(0, _GROUP, zrow, 0)
        pltpu.sync_copy(gath0, acc.at[rsl, :])

    def run_segment(basis, rows_h, cols_h, vals_h, total_wins):
        per = -(-total_wins // _NUM_SUBCORES)
        ngroups = -(-per // _WPG)
        base = sid * per
        my_nw = jnp.clip(total_wins - base, 0, per)
        lim_off = (total_wins - 1) * _WIN

        def drain_gath(sem):
            pltpu.make_async_copy(
                basis.at[pl.ds(0, _GROUP), :], gath0, sem).wait()

        def drain_idx():
            for _ in range(3):
                pltpu.make_async_copy(
                    vals_h.at[pl.ds(0, _GROUP)], vals_b.at[0], isem).wait()

        def fetch_body(b, g, first):
            if not first:
                drain_gath(ssems[b])
            for j in range(_WPG):
                wj = g * _WPG + j
                woff = jnp.minimum((base + wj) * _WIN, lim_off)
                pltpu.async_copy(
                    rows_h.at[pl.ds(woff, _WIN)], rows_b.at[b, j], isem)
                pltpu.async_copy(
                    cols_h.at[pl.ds(woff, _WIN)], cols_b.at[b, j], isem)
                pltpu.async_copy(
                    vals_h.at[pl.ds(woff, _WIN)],
                    vals_b.at[b, pl.ds(j * _WIN, _WIN)], isem)
            drain_idx()
            for j in range(_WPG):
                ok = (g * _WPG + j) < my_nw

                @pl.when(jnp.logical_not(ok))
                def _():
                    for k in range(_WIN // _LANES):
                        vals_b[b, pl.ds(j * _WIN + k * _LANES, _LANES)] = (
                            zero16f)
            for j in range(_WPG):
                pltpu.async_copy(
                    basis.at[cols_b.at[b, j]],
                    gaths[b].at[pl.ds(j * _WIN, _WIN), :], gsems[b])

        def compute_body(b):
            drain_gath(gsems[b])
            gath = gaths[b]

            def srow16(r0, inner):
                v16 = vals_b[b, pl.ds(r0 * _LANES, _LANES)]
                for l in range(_LANES):
                    r = r0 * _LANES + l
                    vv = jnp.broadcast_to(v16[l], (_LANES,))
                    for c in range(_DH // _LANES):
                        sl = pl.ds(c * _LANES, _LANES)
                        gath[r, sl] = gath[r, sl] * vv
                return inner

            lax.fori_loop(0, _GROUP // _LANES, srow16, 0)
            for j in range(_WPG):
                pltpu.async_copy(
                    gath.at[pl.ds(j * _WIN, _WIN), :],
                    acc.at[rows_b.at[b, j]], ssems[b], add=True)

        if ngroups >= 1:
            fetch_body(0, 0, first=True)
        if ngroups >= 2:
            fetch_body(1, 1, first=True)

        def pair(p, carry):
            for b in (0, 1):
                g = 2 * p + b

                @pl.when(g < ngroups)
                def _():
                    compute_body(b)

                @pl.when(g + 2 < ngroups)
                def _():
                    fetch_body(b, g + 2, first=False)
            return carry

        lax.fori_loop(0, (ngroups + 1) // 2, pair, 0)
        drain_gath(ssems[0])
        if ngroups >= 2:
            drain_gath(ssems[1])

    def full_flow(basis, out_r, out_i):
        zero_acc_slice()
        plsc.subcore_barrier()
        run_segment(basis, ct_rows, ct_cols, ct_vals, _CT_WINS)
        run_segment(basis, a0x_rows, a0x_cols, a0x_vals, _A0X_WINS)
        plsc.subcore_barrier()
        pltpu.sync_copy(acc.at[rsl, :], gath0)
        pltpu.sync_copy(gath0, out_r.at[rsl, :])
        zero_acc_slice()
        plsc.subcore_barrier()
        run_segment(basis, a1_rows, a1_cols, a1_vals, _A1_WINS)
        plsc.subcore_barrier()
        pltpu.sync_copy(acc.at[rsl, :], gath0)
        pltpu.sync_copy(gath0, out_i.at[rsl, :])

    @pl.when(cid == 0)
    def _():
        full_flow(basis_lo, out_rl, out_il)

    @pl.when(cid == 1)
    def _():
        full_flow(basis_hi, out_rh, out_ih)


def kernel(atomic_basis, c_tilde_indices, c_tilde_values,
           a_update0_indices, a_update0_values,
           a_update1_indices, a_update1_values):
    idx = c_tilde_indices.astype(jnp.int32)
    ct_rows = idx[1]
    ct_cols = idx[0]
    pad_i = jnp.zeros((_WIN - _CT_TAIL,), jnp.int32)
    a0 = a_update0_indices.astype(jnp.int32)
    a0x_rows = jnp.concatenate([a0[0], ct_rows[_CT_FULL:], pad_i])
    a0x_cols = jnp.concatenate([a0[1], ct_cols[_CT_FULL:], pad_i])
    a0x_vals = jnp.concatenate(
        [a_update0_values, c_tilde_values[_CT_FULL:],
         jnp.zeros((_WIN - _CT_TAIL,), jnp.float32)])
    a1 = a_update1_indices.astype(jnp.int32)

    basis_lo = atomic_basis[:, :_DH]
    basis_hi = atomic_basis[:, _DH:]

    out_rl, out_rh, out_il, out_ih = _sc_spmm(
        basis_lo, basis_hi, ct_rows, ct_cols, c_tilde_values,
        a0x_rows, a0x_cols, a0x_vals, a1[0], a1[1], a_update1_values)
    real_out = jnp.concatenate([out_rl, out_rh], axis=1)
    imag_out = jnp.concatenate([out_il, out_ih], axis=1)
    return (real_out, imag_out)


# speedup vs baseline: 12.9280x; 1.1590x over previous
import functools

import jax
import jax.numpy as jnp
from jax import lax
from jax.experimental import pallas as pl
from jax.experimental.pallas import tpu as pltpu
from jax.experimental.pallas import tpu_sc as plsc

N = 16384
D = 64
_NNZ_C = 268435
_NNZ_U = 65536

_NUM_SUBCORES = 16
_LANES = 16
_DH = D // 2

_WIN = 128
_WPG = 8
_GROUP = _WIN * _WPG
_ROWS_PER_TILE = N // _NUM_SUBCORES

_CT_WINS = _NNZ_C // _WIN
_CT_FULL = _CT_WINS * _WIN
_CT_TAIL = _NNZ_C - _CT_FULL
_A0X_LEN = _NNZ_U + _WIN
_A0X_WINS = _A0X_LEN // _WIN
_A1_WINS = _NNZ_U // _WIN

_mesh = plsc.VectorSubcoreMesh(core_axis_name="c", subcore_axis_name="s")


@functools.partial(
    pl.kernel,
    out_type=(
        jax.ShapeDtypeStruct((N, D), jnp.float32),
        jax.ShapeDtypeStruct((N, D), jnp.float32),
    ),
    mesh=_mesh,
    scratch_types=(
        pltpu.VMEM((2, _WPG, _WIN), jnp.int32),
        pltpu.VMEM((2, _WPG, _WIN), jnp.int32),
        pltpu.VMEM((2, _GROUP), jnp.float32),
        pltpu.VMEM((_GROUP, _DH), jnp.float32),
        pltpu.VMEM((_GROUP, _DH), jnp.float32),
        pltpu.VMEM_SHARED((N, _DH), jnp.float32),
        pltpu.SemaphoreType.DMA,
        pltpu.SemaphoreType.DMA,
        pltpu.SemaphoreType.DMA,
        pltpu.SemaphoreType.DMA,
        pltpu.SemaphoreType.DMA,
    ),
    compiler_params=pltpu.CompilerParams(use_tc_tiling_on_sc=False),
)
def _sc_spmm(basis_lo, basis_hi, ct_idx, ct_vals,
             a0x_rows, a0x_cols, a0x_vals, a1_idx, a1_vals,
             out_r, out_i,
             rows_b, cols_b, vals_b, gath0, gath1, acc,
             gsem0, gsem1, ssem0, ssem1, isem):
    cid = lax.axis_index("c")
    sid = lax.axis_index("s")
    rsl = pl.ds(sid * _ROWS_PER_TILE, _ROWS_PER_TILE)
    zero16f = jnp.zeros((_LANES,), jnp.float32)
    gaths = (gath0, gath1)
    gsems = (gsem0, gsem1)
    ssems = (ssem0, ssem1)

    def zero_acc_slice():
        def zrow(r, carry):
            for c in range(_DH // _LANES):
                gath0[r, pl.ds(c * _LANES, _LANES)] = zero16f
            return carry

        lax.fori_loop(0, _GROUP, zrow, 0)
        pltpu.sync_copy(gath0, acc.at[rsl, :])

    def run_segment(basis, rows_h, cols_h, vals_h, total_wins):
        per = -(-total_wins // _NUM_SUBCORES)
        ngroups = -(-per // _WPG)
        base = sid * per
        my_nw = jnp.clip(total_wins - base, 0, per)
        lim_off = (total_wins - 1) * _WIN

        def drain_gath(sem):
            pltpu.make_async_copy(
                basis.at[pl.ds(0, _GROUP), :], gath0, sem).wait()

        def drain_idx():
            for _ in range(3):
                pltpu.make_async_copy(
                    vals_h.at[pl.ds(0, _GROUP)], vals_b.at[0], isem).wait()

        def fetch_body(b, g, first):
            if not first:
                drain_gath(ssems[b])
            for j in range(_WPG):
                wj = g * _WPG + j
                woff = jnp.minimum((base + wj) * _WIN, lim_off)
                pltpu.async_copy(
                    rows_h.at[pl.ds(woff, _WIN)], rows_b.at[b, j], isem)
                pltpu.async_copy(
                    cols_h.at[pl.ds(woff, _WIN)], cols_b.at[b, j], isem)
                pltpu.async_copy(
                    vals_h.at[pl.ds(woff, _WIN)],
                    vals_b.at[b, pl.ds(j * _WIN, _WIN)], isem)
            drain_idx()
            for j in range(_WPG):
                ok = (g * _WPG + j) < my_nw

                @pl.when(jnp.logical_not(ok))
                def _():
                    for k in range(_WIN // _LANES):
                        vals_b[b, pl.ds(j * _WIN + k * _LANES, _LANES)] = (
                            zero16f)
            for j in range(_WPG):
                pltpu.async_copy(
                    basis.at[cols_b.at[b, j]],
                    gaths[b].at[pl.ds(j * _WIN, _WIN), :], gsems[b])

        def compute_body(b):
            drain_gath(gsems[b])
            gath = gaths[b]

            def srow16(r0, inner):
                v16 = vals_b[b, pl.ds(r0 * _LANES, _LANES)]
                for l in range(_LANES):
                    r = r0 * _LANES + l
                    vv = jnp.broadcast_to(v16[l], (_LANES,))
                    for c in range(_DH // _LANES):
                        sl = pl.ds(c * _LANES, _LANES)
                        gath[r, sl] = gath[r, sl] * vv
                return inner

            lax.fori_loop(0, _GROUP // _LANES, srow16, 0)
            for j in range(_WPG):
                pltpu.async_copy(
                    gath.at[pl.ds(j * _WIN, _WIN), :],
                    acc.at[rows_b.at[b, j]], ssems[b], add=True)

        if ngroups >= 1:
            fetch_body(0, 0, first=True)
        if ngroups >= 2:
            fetch_body(1, 1, first=True)

        def pair(p, carry):
            for b in (0, 1):
                g = 2 * p + b

                @pl.when(g < ngroups)
                def _():
                    compute_body(b)

                @pl.when(g + 2 < ngroups)
                def _():
                    fetch_body(b, g + 2, first=False)
            return carry

        lax.fori_loop(0, (ngroups + 1) // 2, pair, 0)
        drain_gath(ssems[0])
        if ngroups >= 2:
            drain_gath(ssems[1])

    def full_flow(basis, colofs):
        csl = pl.ds(colofs, _DH)
        zero_acc_slice()
        plsc.subcore_barrier()
        run_segment(basis, ct_idx.at[1], ct_idx.at[0], ct_vals, _CT_WINS)
        run_segment(basis, a0x_rows, a0x_cols, a0x_vals, _A0X_WINS)
        plsc.subcore_barrier()
        pltpu.sync_copy(acc.at[rsl, :], gath0)
        pltpu.sync_copy(gath0, out_r.at[rsl, csl])
        zero_acc_slice()
        plsc.subcore_barrier()
        run_segment(basis, a1_idx.at[0], a1_idx.at[1], a1_vals, _A1_WINS)
        plsc.subcore_barrier()
        pltpu.sync_copy(acc.at[rsl, :], gath0)
        pltpu.sync_copy(gath0, out_i.at[rsl, csl])

    @pl.when(cid == 0)
    def _():
        full_flow(basis_lo, 0)

    @pl.when(cid == 1)
    def _():
        full_flow(basis_hi, _DH)


def kernel(atomic_basis, c_tilde_indices, c_tilde_values,
           a_update0_indices, a_update0_values,
           a_update1_indices, a_update1_values):
    pad_i = jnp.zeros((_WIN - _CT_TAIL,), jnp.int32)
    a0x_rows = jnp.concatenate(
        [a_update0_indices[0], c_tilde_indices[1, _CT_FULL:], pad_i])
    a0x_cols = jnp.concatenate(
        [a_update0_indices[1], c_tilde_indices[0, _CT_FULL:], pad_i])
    a0x_vals = jnp.concatenate(
        [a_update0_values, c_tilde_values[_CT_FULL:],
         jnp.zeros((_WIN - _CT_TAIL,), jnp.float32)])

    basis_lo = atomic_basis[:, :_DH]
    basis_hi = atomic_basis[:, _DH:]

    real_out, imag_out = _sc_spmm(
        basis_lo, basis_hi, c_tilde_indices, c_tilde_values,
        a0x_rows, a0x_cols, a0x_vals, a_update1_indices, a_update1_values)
    return (real_out, imag_out)


# speedup vs baseline: 14.5804x; 1.1278x over previous
import functools

import jax
import jax.numpy as jnp
from jax import lax
from jax.experimental import pallas as pl
from jax.experimental.pallas import tpu as pltpu
from jax.experimental.pallas import tpu_sc as plsc

N = 16384
D = 64
_NNZ_C = 268435
_NNZ_U = 65536

_NUM_SUBCORES = 16
_LANES = 16
_DH = D // 2

_WIN = 128
_WPG = 4
_GROUP = _WIN * _WPG
_ROWS_PER_TILE = N // _NUM_SUBCORES

_CT_WINS = _NNZ_C // _WIN
_CT_FULL = _CT_WINS * _WIN
_CT_TAIL = _NNZ_C - _CT_FULL
_A0X_LEN = _NNZ_U + _WIN
_A0X_WINS = _A0X_LEN // _WIN
_A1_WINS = _NNZ_U // _WIN

_mesh = plsc.VectorSubcoreMesh(core_axis_name="c", subcore_axis_name="s")


@functools.partial(
    pl.kernel,
    out_type=(
        jax.ShapeDtypeStruct((N, D), jnp.float32),
        jax.ShapeDtypeStruct((N, D), jnp.float32),
    ),
    mesh=_mesh,
    scratch_types=(
        pltpu.VMEM((3, _WPG, _WIN), jnp.int32),
        pltpu.VMEM((3, _WPG, _WIN), jnp.int32),
        pltpu.VMEM((3, _GROUP), jnp.float32),
        pltpu.VMEM((_GROUP, _DH), jnp.float32),
        pltpu.VMEM((_GROUP, _DH), jnp.float32),
        pltpu.VMEM((_GROUP, _DH), jnp.float32),
        pltpu.VMEM_SHARED((N, _DH), jnp.float32),
        pltpu.SemaphoreType.DMA,
        pltpu.SemaphoreType.DMA,
        pltpu.SemaphoreType.DMA,
        pltpu.SemaphoreType.DMA,
        pltpu.SemaphoreType.DMA,
        pltpu.SemaphoreType.DMA,
        pltpu.SemaphoreType.DMA,
        pltpu.SemaphoreType.DMA,
        pltpu.SemaphoreType.DMA,
    ),
    compiler_params=pltpu.CompilerParams(use_tc_tiling_on_sc=False),
)
def _sc_spmm(basis_lo, basis_hi, ct_idx, ct_vals,
             a0x_rows, a0x_cols, a0x_vals, a1_idx, a1_vals,
             out_r, out_i,
             rows_b, cols_b, vals_b, gath0, gath1, gath2, acc,
             gsem0, gsem1, gsem2, ssem0, ssem1, ssem2,
             isem0, isem1, isem2):
    cid = lax.axis_index("c")
    sid = lax.axis_index("s")
    rsl = pl.ds(sid * _ROWS_PER_TILE, _ROWS_PER_TILE)
    zero16f = jnp.zeros((_LANES,), jnp.float32)
    gaths = (gath0, gath1, gath2)
    gsems = (gsem0, gsem1, gsem2)
    ssems = (ssem0, ssem1, ssem2)
    isems = (isem0, isem1, isem2)

    def zero_acc_slice():
        def zrow(r, carry):
            for c in range(_DH // _LANES):
                gath0[r, pl.ds(c * _LANES, _LANES)] = zero16f
            return carry

        lax.fori_loop(0, _GROUP, zrow, 0)
        for h in range(_ROWS_PER_TILE // _GROUP):
            pltpu.sync_copy(
                gath0,
                acc.at[pl.ds(sid * _ROWS_PER_TILE + h * _GROUP, _GROUP), :])

    def run_segment(basis, rows_h, cols_h, vals_h, total_wins):
        per = -(-total_wins // _NUM_SUBCORES)
        ngroups = -(-per // _WPG)
        base = sid * per
        my_nw = jnp.clip(total_wins - base, 0, per)
        lim_off = (total_wins - 1) * _WIN

        def drain_gath(sem):
            pltpu.make_async_copy(
                basis.at[pl.ds(0, _GROUP), :], gath0, sem).wait()

        def drain_idx(b):
            for _ in range(3):
                pltpu.make_async_copy(
                    vals_h.at[pl.ds(0, _GROUP)], vals_b.at[b], isems[b]).wait()

        def stage_idx(b, g, may_drain):
            if may_drain:
                @pl.when(g >= 3)
                def _():
                    drain_gath(ssems[b])
            for j in range(_WPG):
                wj = g * _WPG + j
                woff = jnp.minimum((base + wj) * _WIN, lim_off)
                pltpu.async_copy(
                    rows_h.at[pl.ds(woff, _WIN)], rows_b.at[b, j], isems[b])
                pltpu.async_copy(
                    cols_h.at[pl.ds(woff, _WIN)], cols_b.at[b, j], isems[b])
                pltpu.async_copy(
                    vals_h.at[pl.ds(woff, _WIN)],
                    vals_b.at[b, pl.ds(j * _WIN, _WIN)], isems[b])

        def stage_gath(b, g):
            drain_idx(b)
            for j in range(_WPG):
                ok = (g * _WPG + j) < my_nw

                @pl.when(jnp.logical_not(ok))
                def _():
                    for k in range(_WIN // _LANES):
                        vals_b[b, pl.ds(j * _WIN + k * _LANES, _LANES)] = (
                            zero16f)
            for j in range(_WPG):
                pltpu.async_copy(
                    basis.at[cols_b.at[b, j]],
                    gaths[b].at[pl.ds(j * _WIN, _WIN), :], gsems[b])

        def stage_comp(b):
            drain_gath(gsems[b])
            gath = gaths[b]

            def srow16(r0, inner):
                v16 = vals_b[b, pl.ds(r0 * _LANES, _LANES)]
                for l in range(_LANES):
                    r = r0 * _LANES + l
                    vv = jnp.broadcast_to(v16[l], (_LANES,))
                    for c in range(_DH // _LANES):
                        sl = pl.ds(c * _LANES, _LANES)
                        gath[r, sl] = gath[r, sl] * vv
                return inner

            lax.fori_loop(0, _GROUP // _LANES, srow16, 0)
            for j in range(_WPG):
                pltpu.async_copy(
                    gath.at[pl.ds(j * _WIN, _WIN), :],
                    acc.at[rows_b.at[b, j]], ssems[b], add=True)

        if ngroups >= 1:
            stage_idx(0, 0, may_drain=False)
        if ngroups >= 2:
            stage_idx(1, 1, may_drain=False)
        stage_gath(0, 0)

        def triple(p, carry):
            for o in (0, 1, 2):
                g = 3 * p + o

                @pl.when(g + 1 < ngroups)
                def _():
                    stage_gath((o + 1) % 3, g + 1)

                @pl.when(g < ngroups)
                def _():
                    stage_comp(o)

                @pl.when(g + 2 < ngroups)
                def _():
                    stage_idx((o + 2) % 3, g + 2, may_drain=True)
            return carry

        lax.fori_loop(0, -(-ngroups // 3), triple, 0)
        for k in range(max(0, ngroups - 3), ngroups):
            drain_gath(ssems[k % 3])

    def writeback(out, colofs):
        for h in range(_ROWS_PER_TILE // _GROUP):
            row0 = sid * _ROWS_PER_TILE + h * _GROUP
            buf = gaths[h % 2]
            pltpu.sync_copy(acc.at[pl.ds(row0, _GROUP), :], buf)
            pltpu.sync_copy(buf, out.at[pl.ds(row0, _GROUP),
                                        pl.ds(colofs, _DH)])

    def full_flow(basis, colofs):
        zero_acc_slice()
        plsc.subcore_barrier()
        run_segment(basis, ct_idx.at[1], ct_idx.at[0], ct_vals, _CT_WINS)
        run_segment(basis, a0x_rows, a0x_cols, a0x_vals, _A0X_WINS)
        plsc.subcore_barrier()
        writeback(out_r, colofs)
        zero_acc_slice()
        plsc.subcore_barrier()
        run_segment(basis, a1_idx.at[0], a1_idx.at[1], a1_vals, _A1_WINS)
        plsc.subcore_barrier()
        writeback(out_i, colofs)

    @pl.when(cid == 0)
    def _():
        full_flow(basis_lo, 0)

    @pl.when(cid == 1)
    def _():
        full_flow(basis_hi, _DH)


def kernel(atomic_basis, c_tilde_indices, c_tilde_values,
           a_update0_indices, a_update0_values,
           a_update1_indices, a_update1_values):
    pad_i = jnp.zeros((_WIN - _CT_TAIL,), jnp.int32)
    a0x_rows = jnp.concatenate(
        [a_update0_indices[0], c_tilde_indices[1, _CT_FULL:], pad_i])
    a0x_cols = jnp.concatenate(
        [a_update0_indices[1], c_tilde_indices[0, _CT_FULL:], pad_i])
    a0x_vals = jnp.concatenate(
        [a_update0_values, c_tilde_values[_CT_FULL:],
         jnp.zeros((_WIN - _CT_TAIL,), jnp.float32)])

    basis_lo = atomic_basis[:, :_DH]
    basis_hi = atomic_basis[:, _DH:]

    real_out, imag_out = _sc_spmm(
        basis_lo, basis_hi, c_tilde_indices, c_tilde_values,
        a0x_rows, a0x_cols, a0x_vals, a_update1_indices, a_update1_values)
    return (real_out, imag_out)
